# baseline (speedup 1.0000x reference)
import functools

import jax
import jax.numpy as jnp
from jax import lax
from jax.experimental import pallas as pl
from jax.experimental.pallas import tpu as pltpu
from jax.experimental.pallas import tpu_sc as plsc

NC = 2
NS = 16
NW = NC * NS
L = 16


def _pick_chunk(per_w):
    for ch in range(128, 7, -8):
        if per_w % ch == 0:
            return ch
    raise ValueError(f"no valid chunk size for {per_w} edges per tile")


def _make_sc_agg(N, F, E, with_deg=False):
    assert E % NW == 0 and N % NS == 0 and F % L == 0
    per_w = E // NW
    ch = _pick_chunk(per_w)
    n_chunks = per_w // ch
    nbuf = 3 if with_deg else 4
    nidx = 8 if nbuf % 2 == 0 else 6
    assert nidx % nbuf == 0 and nidx > 5 and n_chunks >= nidx
    rpt = (N // NS) // 8 * 8
    tail = N - NS * rpt
    assert tail % 8 == 0

    mesh = plsc.VectorSubcoreMesh(core_axis_name="c", subcore_axis_name="s")

    out_type = [jax.ShapeDtypeStruct((NC * N, F), jnp.float32)]
    scratch = (
        [pltpu.VMEM((ch,), jnp.int32) for _ in range(2 * nidx)]
        + [pltpu.VMEM((ch, F), jnp.float32) for _ in range(nbuf)]
        + [pltpu.VMEM_SHARED((N, F), jnp.float32)]
        + [pltpu.SemaphoreType.DMA for _ in range(nidx + 2 * nbuf)]
    )
    if with_deg:
        out_type.append(jax.ShapeDtypeStruct((NW * N,), jnp.float32))
        scratch.append(pltpu.VMEM((N,), jnp.float32))

    def body(y_h, src_h, dst_h, *rest):
        it = iter(rest)
        out_h = next(it)
        deg_h = next(it) if with_deg else None
        sidx = [next(it) for _ in range(nidx)]
        didx = [next(it) for _ in range(nidx)]
        rows = [next(it) for _ in range(nbuf)]
        acc = next(it)
        isem = [next(it) for _ in range(nidx)]
        gsem = [next(it) for _ in range(nbuf)]
        ssem = [next(it) for _ in range(nbuf)]
        deg_v = next(it) if with_deg else None
        c = lax.axis_index("c")
        s = lax.axis_index("s")
        w = c * NS + s

        def zstore(i, _):
            rows[0][i // (F // L), pl.ds((i % (F // L)) * L, L)] = (
                jnp.zeros((L,), jnp.float32))
            return 0
        lax.fori_loop(0, ch * (F // L), zstore, 0)
        if with_deg:
            def zdeg(i, _):
                deg_v[pl.ds(i * L, L)] = jnp.zeros((L,), jnp.float32)
                return 0
            lax.fori_loop(0, N // L, zdeg, 0)
        ones = jnp.ones((L,), jnp.float32)

        def deg_add(q):
            if with_deg:
                def dstep(j, _):
                    dv = didx[q][pl.ds(j * L, L)]
                    plsc.addupdate_scatter(deg_v, [dv], ones)
                    return 0
                lax.fori_loop(0, ch // L, dstep, 0)
        row0 = s * rpt

        def zero_rows(start, nrows):
            done = 0
            while done < nrows:
                nz = min(ch, nrows - done)
                pltpu.sync_copy(rows[0].at[pl.ds(0, nz)],
                                acc.at[pl.ds(start + done, nz)])
                done += nz
        zero_rows(row0, rpt)
        if tail:
            @pl.when(s == NS - 1)
            def _():
                zero_rows(NS * rpt, tail)

        def fetch(ci, q):
            off = w * per_w + ci * ch
            pltpu.async_copy(src_h.at[pl.ds(off, ch)], sidx[q], isem[q])
            pltpu.async_copy(dst_h.at[pl.ds(off, ch)], didx[q], isem[q])

        def fetch_wait(ci, q):
            off = w * per_w + ci * ch
            pltpu.make_async_copy(src_h.at[pl.ds(off, ch)], sidx[q],
                                  isem[q]).wait()
            pltpu.make_async_copy(dst_h.at[pl.ds(off, ch)], didx[q],
                                  isem[q]).wait()

        def gather(ci, q, b):
            pltpu.async_copy(y_h.at[sidx[q]], rows[b], gsem[b])

        def gather_wait(ci, q, b):
            pltpu.make_async_copy(y_h.at[sidx[q]], rows[b], gsem[b]).wait()

        def scat(ci, q, b):
            pltpu.async_copy(rows[b], acc.at[didx[q]], ssem[b], add=True)

        def scat_wait(ci, q, b):
            pltpu.make_async_copy(rows[b], acc.at[didx[q]], ssem[b]).wait()

        for q in range(5):
            fetch(q, q)
        fetch_wait(0, 0)
        fetch_wait(1, 1)
        gather(0, 0, 0)
        gather(1, 1, 1)
        deg_add(0)
        deg_add(1)

        plsc.subcore_barrier()

        def step(ci, s0, first):
            q2 = (s0 + 2) % nidx
            b2 = (s0 + 2) % nbuf
            q0 = s0 % nidx
            b0 = s0 % nbuf
            fetch_wait(ci + 2, q2)
            deg_add(q2)
            if not first:
                scat_wait(ci + 2 - nbuf,
                          (s0 + 2 - nbuf) % nidx, b2)
            gather(ci + 2, q2, b2)
            gather_wait(ci, q0, b0)
            scat(ci, q0, b0)
            if not (isinstance(ci, int) and ci + 5 >= n_chunks):
                fetch(ci + 5, (s0 + 5) % nidx)

        step(0, 0, True)
        step(1, 1, 1 < nbuf - 2)

        n_main = (n_chunks - 7) // nidx

        def loop_body(g, _):
            base = 2 + g * nidx
            for k in range(nidx):
                step(base + k, (2 + k) % nidx, False)
            return 0
        if n_main > 0:
            lax.fori_loop(0, n_main, loop_body, 0)
        for ci in range(2 + n_main * nidx, n_chunks - 2):
            step(ci, ci % nidx, False)

        for ci in (n_chunks - 2, n_chunks - 1):
            gather_wait(ci, ci % nidx, ci % nbuf)
            scat(ci, ci % nidx, ci % nbuf)
        for ci in range(n_chunks - nbuf, n_chunks):
            scat_wait(ci, ci % nidx, ci % nbuf)

        plsc.subcore_barrier()
        pltpu.sync_copy(acc.at[pl.ds(row0, rpt)],
                        out_h.at[pl.ds(c * N + row0, rpt)])
        if tail:
            @pl.when(s == NS - 1)
            def _():
                pltpu.sync_copy(acc.at[pl.ds(NS * rpt, tail)],
                                out_h.at[pl.ds(c * N + NS * rpt, tail)])
        if with_deg:
            pltpu.sync_copy(deg_v, deg_h.at[pl.ds(w * N, N)])

    k = pl.kernel(body, out_type=out_type, mesh=mesh,
                  compiler_params=pltpu.CompilerParams(
                      needs_layout_passes=False),
                  scratch_types=scratch)

    def call(y, src, dst):
        res = k(y, src, dst)
        return res if with_deg else res[0]
    return call


def _make_sc_deg(N, E):
    per_w = E // NW
    ch = _pick_chunk(per_w)
    n_chunks = per_w // ch
    mesh = plsc.VectorSubcoreMesh(core_axis_name="c", subcore_axis_name="s")

    def body(dst_h, deg_h, dst_v, deg_v):
        w = lax.axis_index("c") * NS + lax.axis_index("s")

        def zdeg(i, _):
            deg_v[pl.ds(i * L, L)] = jnp.zeros((L,), jnp.float32)
            return 0
        lax.fori_loop(0, N // L, zdeg, 0)
        pltpu.sync_copy(dst_h.at[w], dst_v)
        ones = jnp.ones((L,), jnp.float32)

        def dstep(i, _):
            dv = dst_v[i // (ch // L), pl.ds((i % (ch // L)) * L, L)]
            plsc.addupdate_scatter(deg_v, [dv], ones)
            return 0
        lax.fori_loop(0, n_chunks * (ch // L), dstep, 0)
        pltpu.sync_copy(deg_v, deg_h.at[pl.ds(w * N, N)])

    k = pl.kernel(
        body, out_type=[jax.ShapeDtypeStruct((NW * N,), jnp.float32)],
        mesh=mesh,
        compiler_params=pltpu.CompilerParams(needs_layout_passes=False),
        scratch_types=[pltpu.VMEM((n_chunks, ch), jnp.int32),
                       pltpu.VMEM((N,), jnp.float32)])

    def call(dst):
        return k(dst.reshape(NW, n_chunks, ch))[0]
    return call


def _mm_bias_body(x_ref, W_ref, b_ref, o_ref):
    o_ref[...] = (jnp.dot(x_ref[...], W_ref[...],
                          preferred_element_type=jnp.float32) + b_ref[...])


def _tc1_body(a0_ref, a1_ref, degp_ref, s1_ref, W1l_ref,
              g1_ref, be1_ref, W2l_ref, h1_ref, y2_ref, dinv_ref):
    deg = jnp.sum(degp_ref[...], axis=0, keepdims=True)
    dinv = 1.0 / jnp.maximum(deg, 1.0)
    dinv_ref[...] = dinv
    mean = (a0_ref[...] + a1_ref[...]) * dinv.T
    pre = (jnp.dot(mean, W1l_ref[...], preferred_element_type=jnp.float32)
           + s1_ref[...])
    m = jnp.mean(pre, axis=0, keepdims=True)
    cen = pre - m
    v = jnp.mean(cen * cen, axis=0, keepdims=True)
    h = jnp.maximum(cen * (g1_ref[...] * jax.lax.rsqrt(v + 1e-5))
                    + be1_ref[...], 0.0)
    h1_ref[...] = h
    y2_ref[...] = jnp.dot(h, W2l_ref[...],
                          preferred_element_type=jnp.float32)


def _tc2_body(a0_ref, a1_ref, dinv_ref, s2_ref,
              g2_ref, be2_ref, h2_ref):
    mean = (a0_ref[...] + a1_ref[...]) * dinv_ref[...].T
    pre = mean + s2_ref[...]
    m = jnp.mean(pre, axis=0, keepdims=True)
    cen = pre - m
    v = jnp.mean(cen * cen, axis=0, keepdims=True)
    h2_ref[...] = jnp.maximum(cen * (g2_ref[...] * jax.lax.rsqrt(v + 1e-5))
                              + be2_ref[...], 0.0)


def _tc3_body(a0_ref, a1_ref, dinv_ref, s3_ref, W3l_ref, out_ref):
    mean = (a0_ref[...] + a1_ref[...]) * dinv_ref[...].T
    out_ref[...] = (jnp.dot(mean, W3l_ref[...],
                            preferred_element_type=jnp.float32)
                    + s3_ref[...])


def kernel(x, edge_index, W1_l, b1, W1_r, g1, be1, W2_l, b2, W2_r,
           g2, be2, W3_l, b3, W3_r):
    N, D = x.shape
    E = edge_index.shape[1]
    H = W1_l.shape[1]
    H2 = W2_l.shape[1]
    OUT = W3_l.shape[1]
    src = edge_index[0]
    dst = edge_index[1]

    f32 = jnp.float32
    sds = jax.ShapeDtypeStruct

    def mm_bias(a, W, b):
        fo = W.shape[1]
        return pl.pallas_call(
            _mm_bias_body, out_shape=sds((N, fo), f32),
        )(a, W, b.reshape(1, fo))

    agg1, degp = _make_sc_agg(N, D, E, with_deg=True)(x, src, dst)
    s1 = mm_bias(x, W1_r, b1)
    degp = degp.reshape(NW, N)

    h1, y2, dinv = pl.pallas_call(
        _tc1_body,
        out_shape=[sds((N, H), f32), sds((N, H2), f32), sds((1, N), f32)],
    )(agg1[:N], agg1[N:], degp, s1, W1_l,
      g1.reshape(1, H), be1.reshape(1, H), W2_l)

    agg2 = _make_sc_agg(N, H2, E)(y2, src, dst)
    s2 = mm_bias(h1, W2_r, b2)
    h2 = pl.pallas_call(
        _tc2_body,
        out_shape=sds((N, H2), f32),
    )(agg2[:N], agg2[N:], dinv, s2,
      g2.reshape(1, H2), be2.reshape(1, H2))

    agg3 = _make_sc_agg(N, H2, E)(h2, src, dst)
    s3 = mm_bias(h2, W3_r, b3)
    out = pl.pallas_call(
        _tc3_body,
        out_shape=sds((N, OUT), f32),
    )(agg3[:N], agg3[N:], dinv, s3, W3_l)
    return out


# speedup vs baseline: 1.0008x; 1.0008x over previous
import jax
import jax.numpy as jnp
from jax import lax
from jax.experimental import pallas as pl
from jax.experimental.pallas import tpu as pltpu
from jax.experimental.pallas import tpu_sc as plsc

NC = 2
NS = 16
NW = NC * NS
L = 16


def _pick_chunk(per_w):
    for ch in range(128, 7, -8):
        if per_w % ch == 0:
            return ch
    raise ValueError(f"no valid chunk size for {per_w} edges per tile")


def _make_sc_agg(N, F, E, with_deg=False):
    assert E % NW == 0 and N % NS == 0 and F % L == 0
    per_w = E // NW
    ch = _pick_chunk(per_w)
    n_chunks = per_w // ch
    nbuf = 3 if with_deg else 4
    nidx = 8 if nbuf % 2 == 0 else 6
    assert nidx % nbuf == 0 and nidx > 5 and n_chunks >= nidx
    rpt = (N // NS) // 8 * 8
    tail = N - NS * rpt
    assert tail % 8 == 0

    mesh = plsc.VectorSubcoreMesh(core_axis_name="c", subcore_axis_name="s")

    out_type = [jax.ShapeDtypeStruct((NC * N, F), jnp.float32)]
    scratch = (
        [pltpu.VMEM((ch,), jnp.int32) for _ in range(2 * nidx)]
        + [pltpu.VMEM((ch, F), jnp.float32) for _ in range(nbuf)]
        + [pltpu.VMEM_SHARED((N, F), jnp.float32)]
        + [pltpu.SemaphoreType.DMA for _ in range(nidx + 2 * nbuf)]
    )
    if with_deg:
        out_type.append(jax.ShapeDtypeStruct((NW * N,), jnp.float32))
        scratch.append(pltpu.VMEM((N,), jnp.float32))

    def body(y_h, src_h, dst_h, *rest):
        it = iter(rest)
        out_h = next(it)
        deg_h = next(it) if with_deg else None
        sidx = [next(it) for _ in range(nidx)]
        didx = [next(it) for _ in range(nidx)]
        rows = [next(it) for _ in range(nbuf)]
        acc = next(it)
        isem = [next(it) for _ in range(nidx)]
        gsem = [next(it) for _ in range(nbuf)]
        ssem = [next(it) for _ in range(nbuf)]
        deg_v = next(it) if with_deg else None
        c = lax.axis_index("c")
        s = lax.axis_index("s")
        w = c * NS + s

        def zstore(i, _):
            rows[0][i // (F // L), pl.ds((i % (F // L)) * L, L)] = (
                jnp.zeros((L,), jnp.float32))
            return 0
        lax.fori_loop(0, ch * (F // L), zstore, 0)
        if with_deg:
            def zdeg(i, _):
                deg_v[pl.ds(i * L, L)] = jnp.zeros((L,), jnp.float32)
                return 0
            lax.fori_loop(0, N // L, zdeg, 0)
        ones = jnp.ones((L,), jnp.float32)

        def deg_add(q):
            if with_deg:
                def dstep(j, _):
                    dv = didx[q][pl.ds(j * L, L)]
                    plsc.addupdate_scatter(deg_v, [dv], ones)
                    return 0
                lax.fori_loop(0, ch // L, dstep, 0)
        row0 = s * rpt

        def zero_rows(start, nrows):
            done = 0
            while done < nrows:
                nz = min(ch, nrows - done)
                pltpu.sync_copy(rows[0].at[pl.ds(0, nz)],
                                acc.at[pl.ds(start + done, nz)])
                done += nz
        zero_rows(row0, rpt)
        if tail:
            @pl.when(s == NS - 1)
            def _():
                zero_rows(NS * rpt, tail)

        def fetch(ci, q):
            off = w * per_w + ci * ch
            pltpu.async_copy(src_h.at[pl.ds(off, ch)], sidx[q], isem[q])
            pltpu.async_copy(dst_h.at[pl.ds(off, ch)], didx[q], isem[q])

        def fetch_wait(ci, q):
            off = w * per_w + ci * ch
            pltpu.make_async_copy(src_h.at[pl.ds(off, ch)], sidx[q],
                                  isem[q]).wait()
            pltpu.make_async_copy(dst_h.at[pl.ds(off, ch)], didx[q],
                                  isem[q]).wait()

        def gather(ci, q, b):
            pltpu.async_copy(y_h.at[sidx[q]], rows[b], gsem[b])

        def gather_wait(ci, q, b):
            pltpu.make_async_copy(y_h.at[sidx[q]], rows[b], gsem[b]).wait()

        def scat(ci, q, b):
            pltpu.async_copy(rows[b], acc.at[didx[q]], ssem[b], add=True)

        def scat_wait(ci, q, b):
            pltpu.make_async_copy(rows[b], acc.at[didx[q]], ssem[b]).wait()

        for q in range(5):
            fetch(q, q)
        fetch_wait(0, 0)
        fetch_wait(1, 1)
        gather(0, 0, 0)
        gather(1, 1, 1)
        deg_add(0)
        deg_add(1)

        plsc.subcore_barrier()

        def step(ci, s0, first):
            q2 = (s0 + 2) % nidx
            b2 = (s0 + 2) % nbuf
            q0 = s0 % nidx
            b0 = s0 % nbuf
            fetch_wait(ci + 2, q2)
            deg_add(q2)
            if not first:
                scat_wait(ci + 2 - nbuf,
                          (s0 + 2 - nbuf) % nidx, b2)
            gather(ci + 2, q2, b2)
            gather_wait(ci, q0, b0)
            scat(ci, q0, b0)
            if not (isinstance(ci, int) and ci + 5 >= n_chunks):
                fetch(ci + 5, (s0 + 5) % nidx)

        step(0, 0, True)
        step(1, 1, 1 < nbuf - 2)

        n_main = (n_chunks - 7) // nidx

        def loop_body(g, _):
            base = 2 + g * nidx
            for k in range(nidx):
                step(base + k, (2 + k) % nidx, False)
            return 0
        if n_main > 0:
            lax.fori_loop(0, n_main, loop_body, 0)
        for ci in range(2 + n_main * nidx, n_chunks - 2):
            step(ci, ci % nidx, False)

        for ci in (n_chunks - 2, n_chunks - 1):
            gather_wait(ci, ci % nidx, ci % nbuf)
            scat(ci, ci % nidx, ci % nbuf)
        for ci in range(n_chunks - nbuf, n_chunks):
            scat_wait(ci, ci % nidx, ci % nbuf)

        plsc.subcore_barrier()
        pltpu.sync_copy(acc.at[pl.ds(row0, rpt)],
                        out_h.at[pl.ds(c * N + row0, rpt)])
        if tail:
            @pl.when(s == NS - 1)
            def _():
                pltpu.sync_copy(acc.at[pl.ds(NS * rpt, tail)],
                                out_h.at[pl.ds(c * N + NS * rpt, tail)])
        if with_deg:
            pltpu.sync_copy(deg_v, deg_h.at[pl.ds(w * N, N)])

    k = pl.kernel(body, out_type=out_type, mesh=mesh,
                  compiler_params=pltpu.CompilerParams(
                      needs_layout_passes=False),
                  scratch_types=scratch)

    def call(y, src, dst):
        res = k(y, src, dst)
        return res if with_deg else res[0]
    return call


def _tc1_body(a0_ref, a1_ref, degp_ref, x_ref, W1l_ref, b1_ref, W1r_ref,
              g1_ref, be1_ref, W2l_ref, h1_ref, y2_ref, dinv_ref):
    deg = jnp.sum(degp_ref[...], axis=0, keepdims=True)
    dinv = 1.0 / jnp.maximum(deg, 1.0)
    dinv_ref[...] = dinv
    mean = (a0_ref[...] + a1_ref[...]) * dinv.T
    pre = (jnp.dot(mean, W1l_ref[...], preferred_element_type=jnp.float32)
           + b1_ref[...]
           + jnp.dot(x_ref[...], W1r_ref[...],
                     preferred_element_type=jnp.float32))
    m = jnp.mean(pre, axis=0, keepdims=True)
    cen = pre - m
    v = jnp.mean(cen * cen, axis=0, keepdims=True)
    h = jnp.maximum(cen * (g1_ref[...] * jax.lax.rsqrt(v + 1e-5))
                    + be1_ref[...], 0.0)
    h1_ref[...] = h
    y2_ref[...] = jnp.dot(h, W2l_ref[...],
                          preferred_element_type=jnp.float32)


def _tc2_body(a0_ref, a1_ref, dinv_ref, h1_ref, b2_ref, W2r_ref,
              g2_ref, be2_ref, h2_ref):
    mean = (a0_ref[...] + a1_ref[...]) * dinv_ref[...].T
    pre = (mean + b2_ref[...]
           + jnp.dot(h1_ref[...], W2r_ref[...],
                     preferred_element_type=jnp.float32))
    m = jnp.mean(pre, axis=0, keepdims=True)
    cen = pre - m
    v = jnp.mean(cen * cen, axis=0, keepdims=True)
    h2_ref[...] = jnp.maximum(cen * (g2_ref[...] * jax.lax.rsqrt(v + 1e-5))
                              + be2_ref[...], 0.0)


def _tc3_body(a0_ref, a1_ref, dinv_ref, h2_ref, W3l_ref, b3_ref, W3r_ref,
              out_ref):
    mean = (a0_ref[...] + a1_ref[...]) * dinv_ref[...].T
    out_ref[...] = (jnp.dot(mean, W3l_ref[...],
                            preferred_element_type=jnp.float32)
                    + b3_ref[...]
                    + jnp.dot(h2_ref[...], W3r_ref[...],
                              preferred_element_type=jnp.float32))


def kernel(x, edge_index, W1_l, b1, W1_r, g1, be1, W2_l, b2, W2_r,
           g2, be2, W3_l, b3, W3_r):
    N, D = x.shape
    E = edge_index.shape[1]
    H = W1_l.shape[1]
    H2 = W2_l.shape[1]
    OUT = W3_l.shape[1]
    src = edge_index[0]
    dst = edge_index[1]

    f32 = jnp.float32
    sds = jax.ShapeDtypeStruct

    agg1, degp = _make_sc_agg(N, D, E, with_deg=True)(x, src, dst)
    degp = degp.reshape(NW, N)
    a10 = agg1[:N]
    a11 = agg1[N:]

    h1, y2, dinv = pl.pallas_call(
        _tc1_body,
        out_shape=[sds((N, H), f32), sds((N, H2), f32), sds((1, N), f32)],
    )(a10, a11, degp, x, W1_l, b1.reshape(1, H), W1_r,
      g1.reshape(1, H), be1.reshape(1, H), W2_l)

    agg2 = _make_sc_agg(N, H2, E)(y2, src, dst)
    h2 = pl.pallas_call(
        _tc2_body,
        out_shape=sds((N, H2), f32),
    )(agg2[:N], agg2[N:], dinv, h1, b2.reshape(1, H2), W2_r,
      g2.reshape(1, H2), be2.reshape(1, H2))

    agg3 = _make_sc_agg(N, H2, E)(h2, src, dst)
    out = pl.pallas_call(
        _tc3_body,
        out_shape=sds((N, OUT), f32),
    )(agg3[:N], agg3[N:], dinv, h2, W3_l, b3.reshape(1, OUT), W3_r)
    return out


# speedup vs baseline: 1.0515x; 1.0507x over previous
import jax
import jax.numpy as jnp
from jax import lax
from jax.experimental import pallas as pl
from jax.experimental.pallas import tpu as pltpu
from jax.experimental.pallas import tpu_sc as plsc

NC = 2
NS = 16
NW = NC * NS
L = 16


def _pick_chunk(per_w):
    for ch in range(128, 7, -8):
        if per_w % ch == 0:
            return ch
    raise ValueError(f"no valid chunk size for {per_w} edges per tile")


def _make_sc_agg(N, F, E, with_deg=False):
    assert E % NW == 0 and N % NS == 0 and F % L == 0
    per_w = E // NW
    ch = _pick_chunk(per_w)
    n_chunks = per_w // ch
    nbuf = 3 if with_deg else 4
    nidx = 8 if nbuf % 2 == 0 else 6
    assert nidx % nbuf == 0 and nidx > 5 and n_chunks >= nidx
    rpt = (N // NS) // 8 * 8
    tail = N - NS * rpt
    assert tail % 8 == 0

    mesh = plsc.VectorSubcoreMesh(core_axis_name="c", subcore_axis_name="s")

    out_type = [jax.ShapeDtypeStruct((NC * N, F), jnp.float32)]
    scratch = (
        [pltpu.VMEM((ch,), jnp.int32) for _ in range(2 * nidx)]
        + [pltpu.VMEM((ch, F), jnp.float32) for _ in range(nbuf)]
        + [pltpu.VMEM_SHARED((N, F), jnp.float32)]
        + [pltpu.SemaphoreType.DMA for _ in range(nidx + 2 * nbuf)]
    )
    if with_deg:
        out_type.append(jax.ShapeDtypeStruct((NW * N,), jnp.float32))
        scratch.append(pltpu.VMEM((N,), jnp.float32))

    def body(y_h, src_h, dst_h, *rest):
        it = iter(rest)
        out_h = next(it)
        deg_h = next(it) if with_deg else None
        sidx = [next(it) for _ in range(nidx)]
        didx = [next(it) for _ in range(nidx)]
        rows = [next(it) for _ in range(nbuf)]
        acc = next(it)
        isem = [next(it) for _ in range(nidx)]
        gsem = [next(it) for _ in range(nbuf)]
        ssem = [next(it) for _ in range(nbuf)]
        deg_v = next(it) if with_deg else None
        c = lax.axis_index("c")
        s = lax.axis_index("s")
        w = c * NS + s

        def zstore(i, _):
            rows[0][i // (F // L), pl.ds((i % (F // L)) * L, L)] = (
                jnp.zeros((L,), jnp.float32))
            return 0
        lax.fori_loop(0, ch * (F // L), zstore, 0)
        if with_deg:
            def zdeg(i, _):
                deg_v[pl.ds(i * L, L)] = jnp.zeros((L,), jnp.float32)
                return 0
            lax.fori_loop(0, N // L, zdeg, 0)
        ones = jnp.ones((L,), jnp.float32)

        def deg_add(q):
            if with_deg:
                def dstep(j, _):
                    dv = didx[q][pl.ds(j * L, L)]
                    plsc.addupdate_scatter(deg_v, [dv], ones)
                    return 0
                lax.fori_loop(0, ch // L, dstep, 0)
        row0 = s * rpt

        def zero_rows(start, nrows):
            done = 0
            while done < nrows:
                nz = min(ch, nrows - done)
                pltpu.sync_copy(rows[0].at[pl.ds(0, nz)],
                                acc.at[pl.ds(start + done, nz)])
                done += nz
        zero_rows(row0, rpt)
        if tail:
            @pl.when(s == NS - 1)
            def _():
                zero_rows(NS * rpt, tail)

        def fetch(ci, q):
            off = w * per_w + ci * ch
            pltpu.async_copy(src_h.at[pl.ds(off, ch)], sidx[q], isem[q])
            pltpu.async_copy(dst_h.at[pl.ds(off, ch)], didx[q], isem[q])

        def fetch_wait(ci, q):
            off = w * per_w + ci * ch
            pltpu.make_async_copy(src_h.at[pl.ds(off, ch)], sidx[q],
                                  isem[q]).wait()
            pltpu.make_async_copy(dst_h.at[pl.ds(off, ch)], didx[q],
                                  isem[q]).wait()

        def gather(ci, q, b):
            pltpu.async_copy(y_h.at[sidx[q]], rows[b], gsem[b])

        def gather_wait(ci, q, b):
            pltpu.make_async_copy(y_h.at[sidx[q]], rows[b], gsem[b]).wait()

        def scat(ci, q, b):
            pltpu.async_copy(rows[b], acc.at[didx[q]], ssem[b], add=True)

        def scat_wait(ci, q, b):
            pltpu.make_async_copy(rows[b], acc.at[didx[q]], ssem[b]).wait()

        for q in range(5):
            fetch(q, q)
        fetch_wait(0, 0)
        fetch_wait(1, 1)
        gather(0, 0, 0)
        gather(1, 1, 1)
        deg_add(0)
        deg_add(1)

        plsc.subcore_barrier()

        def step(ci, s0, first):
            q2 = (s0 + 2) % nidx
            b2 = (s0 + 2) % nbuf
            q0 = s0 % nidx
            b0 = s0 % nbuf
            fetch_wait(ci + 2, q2)
            deg_add(q2)
            if not first:
                scat_wait(ci + 2 - nbuf,
                          (s0 + 2 - nbuf) % nidx, b2)
            gather(ci + 2, q2, b2)
            gather_wait(ci, q0, b0)
            scat(ci, q0, b0)
            if not (isinstance(ci, int) and ci + 5 >= n_chunks):
                fetch(ci + 5, (s0 + 5) % nidx)

        step(0, 0, True)
        step(1, 1, 1 < nbuf - 2)

        n_main = (n_chunks - 7) // nidx

        def loop_body(g, _):
            base = 2 + g * nidx
            for k in range(nidx):
                step(base + k, (2 + k) % nidx, False)
            return 0
        if n_main > 0:
            lax.fori_loop(0, n_main, loop_body, 0)
        for ci in range(2 + n_main * nidx, n_chunks - 2):
            step(ci, ci % nidx, False)

        for ci in (n_chunks - 2, n_chunks - 1):
            gather_wait(ci, ci % nidx, ci % nbuf)
            scat(ci, ci % nidx, ci % nbuf)
        for ci in range(n_chunks - nbuf, n_chunks):
            scat_wait(ci, ci % nidx, ci % nbuf)

        plsc.subcore_barrier()
        pltpu.sync_copy(acc.at[pl.ds(row0, rpt)],
                        out_h.at[pl.ds(c * N + row0, rpt)])
        if tail:
            @pl.when(s == NS - 1)
            def _():
                pltpu.sync_copy(acc.at[pl.ds(NS * rpt, tail)],
                                out_h.at[pl.ds(c * N + NS * rpt, tail)])
        if with_deg:
            pltpu.sync_copy(deg_v, deg_h.at[pl.ds(w * N, N)])

    k = pl.kernel(body, out_type=out_type, mesh=mesh,
                  compiler_params=pltpu.CompilerParams(
                      needs_layout_passes=False),
                  scratch_types=scratch)

    def call(y, src, dst):
        res = k(y, src, dst)
        return res if with_deg else res[0]
    return call


def _tc1_body(a_ref, degp_ref, x_ref, W1l_ref, b1_ref, W1r_ref,
              g1_ref, be1_ref, W2l_ref, h1_ref, y2_ref, dinv_ref):
    n = a_ref.shape[0] // 2
    deg = jnp.sum(degp_ref[...], axis=0, keepdims=True)
    dinv = 1.0 / jnp.maximum(deg, 1.0)
    dinv_ref[...] = dinv
    mean = (a_ref[:n] + a_ref[n:]) * dinv.T
    pre = (jnp.dot(mean, W1l_ref[...], preferred_element_type=jnp.float32)
           + b1_ref[...]
           + jnp.dot(x_ref[...], W1r_ref[...],
                     preferred_element_type=jnp.float32))
    m = jnp.mean(pre, axis=0, keepdims=True)
    cen = pre - m
    v = jnp.mean(cen * cen, axis=0, keepdims=True)
    h = jnp.maximum(cen * (g1_ref[...] * jax.lax.rsqrt(v + 1e-5))
                    + be1_ref[...], 0.0)
    h1_ref[...] = h
    y2_ref[...] = jnp.dot(h, W2l_ref[...],
                          preferred_element_type=jnp.float32)


def _tc2_body(a_ref, dinv_ref, h1_ref, b2_ref, W2r_ref,
              g2_ref, be2_ref, h2_ref):
    n = a_ref.shape[0] // 2
    mean = (a_ref[:n] + a_ref[n:]) * dinv_ref[...].T
    pre = (mean + b2_ref[...]
           + jnp.dot(h1_ref[...], W2r_ref[...],
                     preferred_element_type=jnp.float32))
    m = jnp.mean(pre, axis=0, keepdims=True)
    cen = pre - m
    v = jnp.mean(cen * cen, axis=0, keepdims=True)
    h2_ref[...] = jnp.maximum(cen * (g2_ref[...] * jax.lax.rsqrt(v + 1e-5))
                              + be2_ref[...], 0.0)


def _tc3_body(a_ref, dinv_ref, h2_ref, W3l_ref, b3_ref, W3r_ref,
              out_ref):
    n = a_ref.shape[0] // 2
    mean = (a_ref[:n] + a_ref[n:]) * dinv_ref[...].T
    out_ref[...] = (jnp.dot(mean, W3l_ref[...],
                            preferred_element_type=jnp.float32)
                    + b3_ref[...]
                    + jnp.dot(h2_ref[...], W3r_ref[...],
                              preferred_element_type=jnp.float32))


def kernel(x, edge_index, W1_l, b1, W1_r, g1, be1, W2_l, b2, W2_r,
           g2, be2, W3_l, b3, W3_r):
    N, D = x.shape
    E = edge_index.shape[1]
    H = W1_l.shape[1]
    H2 = W2_l.shape[1]
    OUT = W3_l.shape[1]
    src = edge_index[0]
    dst = edge_index[1]

    f32 = jnp.float32
    sds = jax.ShapeDtypeStruct

    agg1, degp = _make_sc_agg(N, D, E, with_deg=True)(x, src, dst)
    degp = degp.reshape(NW, N)

    h1, y2, dinv = pl.pallas_call(
        _tc1_body,
        out_shape=[sds((N, H), f32), sds((N, H2), f32), sds((1, N), f32)],
    )(agg1, degp, x, W1_l, b1.reshape(1, H), W1_r,
      g1.reshape(1, H), be1.reshape(1, H), W2_l)

    agg2 = _make_sc_agg(N, H2, E)(y2, src, dst)
    h2 = pl.pallas_call(
        _tc2_body,
        out_shape=sds((N, H2), f32),
    )(agg2, dinv, h1, b2.reshape(1, H2), W2_r,
      g2.reshape(1, H2), be2.reshape(1, H2))

    agg3 = _make_sc_agg(N, H2, E)(h2, src, dst)
    out = pl.pallas_call(
        _tc3_body,
        out_shape=sds((N, OUT), f32),
    )(agg3, dinv, h2, W3_l, b3.reshape(1, OUT), W3_r)
    return out


# speedup vs baseline: 1.0796x; 1.0268x over previous
import jax
import jax.numpy as jnp
from jax import lax
from jax.experimental import pallas as pl
from jax.experimental.pallas import tpu as pltpu
from jax.experimental.pallas import tpu_sc as plsc

NC = 2
NS = 16
NW = NC * NS
L = 16


def _pick_chunk(per_w):
    for ch in range(128, 7, -8):
        if per_w % ch == 0:
            return ch
    raise ValueError(f"no valid chunk size for {per_w} edges per tile")


def _make_sc_agg(N, F, E, with_deg=False):
    assert E % NW == 0 and N % NS == 0 and F % L == 0
    per_w = E // NW
    ch = _pick_chunk(per_w)
    n_chunks = per_w // ch
    nbuf = 3 if with_deg else 4
    nidx = 8 if nbuf % 2 == 0 else 6
    assert nidx % nbuf == 0 and nidx > 5 and n_chunks >= nidx
    rpt = (N // NS) // 8 * 8
    tail = N - NS * rpt
    assert tail % 8 == 0

    mesh = plsc.VectorSubcoreMesh(core_axis_name="c", subcore_axis_name="s")

    out_type = [jax.ShapeDtypeStruct((NC * N, F), jnp.float32)]
    scratch = (
        [pltpu.VMEM((ch,), jnp.int32) for _ in range(2 * nidx)]
        + [pltpu.VMEM((ch, F), jnp.float32) for _ in range(nbuf)]
        + [pltpu.VMEM_SHARED((N, F), jnp.float32)]
        + [pltpu.SemaphoreType.DMA for _ in range(nidx + 2 * nbuf)]
    )
    if with_deg:
        out_type.append(jax.ShapeDtypeStruct((NW * N,), jnp.float32))
        scratch.append(pltpu.VMEM((N,), jnp.float32))

    def body(y_h, ei_h, *rest):
        it = iter(rest)
        out_h = next(it)
        deg_h = next(it) if with_deg else None
        sidx = [next(it) for _ in range(nidx)]
        didx = [next(it) for _ in range(nidx)]
        rows = [next(it) for _ in range(nbuf)]
        acc = next(it)
        isem = [next(it) for _ in range(nidx)]
        gsem = [next(it) for _ in range(nbuf)]
        ssem = [next(it) for _ in range(nbuf)]
        deg_v = next(it) if with_deg else None
        c = lax.axis_index("c")
        s = lax.axis_index("s")
        w = c * NS + s

        def zstore(i, _):
            rows[0][i // (F // L), pl.ds((i % (F // L)) * L, L)] = (
                jnp.zeros((L,), jnp.float32))
            return 0
        lax.fori_loop(0, ch * (F // L), zstore, 0)
        if with_deg:
            def zdeg(i, _):
                deg_v[pl.ds(i * L, L)] = jnp.zeros((L,), jnp.float32)
                return 0
            lax.fori_loop(0, N // L, zdeg, 0)
        ones = jnp.ones((L,), jnp.float32)

        def deg_add(q):
            if with_deg:
                def dstep(j, _):
                    dv = didx[q][pl.ds(j * L, L)]
                    plsc.addupdate_scatter(deg_v, [dv], ones)
                    return 0
                lax.fori_loop(0, ch // L, dstep, 0)
        row0 = s * rpt

        def zero_rows(start, nrows):
            done = 0
            while done < nrows:
                nz = min(ch, nrows - done)
                pltpu.sync_copy(rows[0].at[pl.ds(0, nz)],
                                acc.at[pl.ds(start + done, nz)])
                done += nz
        zero_rows(row0, rpt)
        if tail:
            @pl.when(s == NS - 1)
            def _():
                zero_rows(NS * rpt, tail)

        def fetch(ci, q):
            off = w * per_w + ci * ch
            pltpu.async_copy(ei_h.at[pl.ds(off, ch)], sidx[q], isem[q])
            pltpu.async_copy(ei_h.at[pl.ds(E + off, ch)], didx[q], isem[q])

        def fetch_wait(ci, q):
            off = w * per_w + ci * ch
            pltpu.make_async_copy(ei_h.at[pl.ds(off, ch)], sidx[q],
                                  isem[q]).wait()
            pltpu.make_async_copy(ei_h.at[pl.ds(E + off, ch)], didx[q],
                                  isem[q]).wait()

        def gather(ci, q, b):
            pltpu.async_copy(y_h.at[sidx[q]], rows[b], gsem[b])

        def gather_wait(ci, q, b):
            pltpu.make_async_copy(y_h.at[sidx[q]], rows[b], gsem[b]).wait()

        def scat(ci, q, b):
            pltpu.async_copy(rows[b], acc.at[didx[q]], ssem[b], add=True)

        def scat_wait(ci, q, b):
            pltpu.make_async_copy(rows[b], acc.at[didx[q]], ssem[b]).wait()

        for q in range(5):
            fetch(q, q)
        fetch_wait(0, 0)
        fetch_wait(1, 1)
        gather(0, 0, 0)
        gather(1, 1, 1)
        deg_add(0)
        deg_add(1)

        plsc.subcore_barrier()

        def step(ci, s0, first):
            q2 = (s0 + 2) % nidx
            b2 = (s0 + 2) % nbuf
            q0 = s0 % nidx
            b0 = s0 % nbuf
            fetch_wait(ci + 2, q2)
            deg_add(q2)
            if not first:
                scat_wait(ci + 2 - nbuf,
                          (s0 + 2 - nbuf) % nidx, b2)
            gather(ci + 2, q2, b2)
            gather_wait(ci, q0, b0)
            scat(ci, q0, b0)
            if not (isinstance(ci, int) and ci + 5 >= n_chunks):
                fetch(ci + 5, (s0 + 5) % nidx)

        step(0, 0, True)
        step(1, 1, 1 < nbuf - 2)

        n_main = (n_chunks - 7) // nidx

        def loop_body(g, _):
            base = 2 + g * nidx
            for k in range(nidx):
                step(base + k, (2 + k) % nidx, False)
            return 0
        if n_main > 0:
            lax.fori_loop(0, n_main, loop_body, 0)
        for ci in range(2 + n_main * nidx, n_chunks - 2):
            step(ci, ci % nidx, False)

        for ci in (n_chunks - 2, n_chunks - 1):
            gather_wait(ci, ci % nidx, ci % nbuf)
            scat(ci, ci % nidx, ci % nbuf)
        for ci in range(n_chunks - nbuf, n_chunks):
            scat_wait(ci, ci % nidx, ci % nbuf)

        plsc.subcore_barrier()
        pltpu.sync_copy(acc.at[pl.ds(row0, rpt)],
                        out_h.at[pl.ds(c * N + row0, rpt)])
        if tail:
            @pl.when(s == NS - 1)
            def _():
                pltpu.sync_copy(acc.at[pl.ds(NS * rpt, tail)],
                                out_h.at[pl.ds(c * N + NS * rpt, tail)])
        if with_deg:
            pltpu.sync_copy(deg_v, deg_h.at[pl.ds(w * N, N)])

    k = pl.kernel(body, out_type=out_type, mesh=mesh,
                  compiler_params=pltpu.CompilerParams(
                      needs_layout_passes=False),
                  scratch_types=scratch)

    def call(y, ei_flat):
        res = k(y, ei_flat)
        return res if with_deg else res[0]
    return call


def _tc1_body(a_ref, degp_ref, x_ref, W1l_ref, b1_ref, W1r_ref,
              g1_ref, be1_ref, W2l_ref, h1_ref, y2_ref, dinv_ref):
    n = a_ref.shape[0] // 2
    deg = jnp.sum(degp_ref[...], axis=0, keepdims=True)
    dinv = 1.0 / jnp.maximum(deg, 1.0)
    dinv_ref[...] = dinv
    mean = (a_ref[:n] + a_ref[n:]) * dinv.T
    pre = (jnp.dot(mean, W1l_ref[...], preferred_element_type=jnp.float32)
           + b1_ref[...]
           + jnp.dot(x_ref[...], W1r_ref[...],
                     preferred_element_type=jnp.float32))
    m = jnp.mean(pre, axis=0, keepdims=True)
    cen = pre - m
    v = jnp.mean(cen * cen, axis=0, keepdims=True)
    h = jnp.maximum(cen * (g1_ref[...] * jax.lax.rsqrt(v + 1e-5))
                    + be1_ref[...], 0.0)
    h1_ref[...] = h
    y2_ref[...] = jnp.dot(h, W2l_ref[...],
                          preferred_element_type=jnp.float32)


def _tc2_body(a_ref, dinv_ref, h1_ref, b2_ref, W2r_ref,
              g2_ref, be2_ref, h2_ref):
    n = a_ref.shape[0] // 2
    mean = (a_ref[:n] + a_ref[n:]) * dinv_ref[...].T
    pre = (mean + b2_ref[...]
           + jnp.dot(h1_ref[...], W2r_ref[...],
                     preferred_element_type=jnp.float32))
    m = jnp.mean(pre, axis=0, keepdims=True)
    cen = pre - m
    v = jnp.mean(cen * cen, axis=0, keepdims=True)
    h2_ref[...] = jnp.maximum(cen * (g2_ref[...] * jax.lax.rsqrt(v + 1e-5))
                              + be2_ref[...], 0.0)


def _tc3_body(a_ref, dinv_ref, h2_ref, W3l_ref, b3_ref, W3r_ref,
              out_ref):
    n = a_ref.shape[0] // 2
    mean = (a_ref[:n] + a_ref[n:]) * dinv_ref[...].T
    out_ref[...] = (jnp.dot(mean, W3l_ref[...],
                            preferred_element_type=jnp.float32)
                    + b3_ref[...]
                    + jnp.dot(h2_ref[...], W3r_ref[...],
                              preferred_element_type=jnp.float32))


def kernel(x, edge_index, W1_l, b1, W1_r, g1, be1, W2_l, b2, W2_r,
           g2, be2, W3_l, b3, W3_r):
    N, D = x.shape
    E = edge_index.shape[1]
    H = W1_l.shape[1]
    H2 = W2_l.shape[1]
    OUT = W3_l.shape[1]
    ei_flat = edge_index.reshape(2 * E)

    f32 = jnp.float32
    sds = jax.ShapeDtypeStruct

    agg1, degp = _make_sc_agg(N, D, E, with_deg=True)(x, ei_flat)
    degp = degp.reshape(NW, N)

    h1, y2, dinv = pl.pallas_call(
        _tc1_body,
        out_shape=[sds((N, H), f32), sds((N, H2), f32), sds((1, N), f32)],
    )(agg1, degp, x, W1_l, b1.reshape(1, H), W1_r,
      g1.reshape(1, H), be1.reshape(1, H), W2_l)

    agg2 = _make_sc_agg(N, H2, E)(y2, ei_flat)
    h2 = pl.pallas_call(
        _tc2_body,
        out_shape=sds((N, H2), f32),
    )(agg2, dinv, h1, b2.reshape(1, H2), W2_r,
      g2.reshape(1, H2), be2.reshape(1, H2))

    agg3 = _make_sc_agg(N, H2, E)(h2, ei_flat)
    out = pl.pallas_call(
        _tc3_body,
        out_shape=sds((N, OUT), f32),
    )(agg3, dinv, h2, W3_l, b3.reshape(1, OUT), W3_r)
    return out
